# baseline (speedup 1.0000x reference)
import functools

import jax
import jax.numpy as jnp
from jax import lax
from jax.experimental import pallas as pl
from jax.experimental.pallas import tpu as pltpu
from jax.experimental.pallas import tpu_sc as plsc

_V = 1000000
_D = 32
_B = 16384
_K = 20

_NC = 2
_NS = 16
_NW = _NC * _NS
_BPW = _B // _NW
_CB = 64
_NCH = _BPW // _CB
_NIDX_ROWS = _CB * _K // 128

_LOG1P_C = (
    2.4200538240037872e-09, 0.999999668889092, -0.49998875344797256,
    0.33316686590823513, -0.24865795250658715, 0.19337563668723085,
    -0.1451751324863907, 0.09470229552014076, -0.04713243998914813,
    0.015144988822244822, -0.0022880009946668264,
)


def _softplus(t):
    e = jnp.exp(-jnp.abs(t))
    p = jnp.full((16,), _LOG1P_C[-1], jnp.float32)
    for c in _LOG1P_C[-2::-1]:
        p = p * e + jnp.float32(c)
    return jnp.maximum(t, jnp.float32(0.0)) + p


def _sc_body(c3, p3, n4, w_in, w_out, out,
             cidx, pidx, nidx, crows, prows, nrows, accv, sem):
    w = lax.axis_index("s") * _NC + lax.axis_index("c")
    iota = lax.iota(jnp.int32, 16)

    def chunk_body(i, acc):
        pltpu.sync_copy(c3.at[w, i], cidx)
        pltpu.sync_copy(p3.at[w, i], pidx)
        pltpu.sync_copy(n4.at[w, i], nidx)
        cps = [pltpu.async_copy(w_in.at[cidx], crows, sem),
               pltpu.async_copy(w_out.at[pidx], prows, sem)]
        for j in range(_NIDX_ROWS):
            cps.append(pltpu.async_copy(
                w_out.at[nidx.at[j]], nrows.at[pl.ds(j * 128, 128)], sem))
        for cp in cps:
            cp.wait()

        def group_body(g, acc):
            bvec = iota + g * 16
            nbase = bvec * _K

            def d_body(d, accs):
                dvec = jnp.full((16,), d, jnp.int32)
                c_d = plsc.load_gather(crows, [bvec, dvec])
                p_d = plsc.load_gather(prows, [bvec, dvec])
                new = [accs[0] + c_d * p_d]
                for k in range(_K):
                    n_d = plsc.load_gather(nrows, [nbase + k, dvec])
                    new.append(accs[k + 1] + c_d * n_d)
                return new

            zero = jnp.zeros((16,), jnp.float32)
            accs = lax.fori_loop(0, _D, d_body, [zero] * (_K + 1))
            total = _softplus(-accs[0])
            for k in range(_K):
                total = total + _softplus(accs[k + 1])
            return acc + total

        return lax.fori_loop(0, _CB // 16, group_body, acc)

    acc = lax.fori_loop(0, _NCH, chunk_body, jnp.zeros((16,), jnp.float32))
    accv[...] = acc
    pltpu.sync_copy(accv, out.at[w])


@functools.partial(jax.jit)
def kernel(centers, positives, negatives, W_in, W_out):
    c3 = centers.reshape(_NW, _NCH, _CB)
    p3 = positives.reshape(_NW, _NCH, _CB)
    n4 = negatives.reshape(_NW, _NCH, _NIDX_ROWS, 128)
    mesh = plsc.VectorSubcoreMesh(core_axis_name="c", subcore_axis_name="s")
    partials = pl.kernel(
        _sc_body,
        mesh=mesh,
        compiler_params=pltpu.CompilerParams(
            needs_layout_passes=False, use_tc_tiling_on_sc=False),
        out_type=jax.ShapeDtypeStruct((_NW, 16), jnp.float32),
        scratch_types=[
            pltpu.VMEM((_CB,), jnp.int32),
            pltpu.VMEM((_CB,), jnp.int32),
            pltpu.VMEM((_NIDX_ROWS, 128), jnp.int32),
            pltpu.VMEM((_CB, _D), jnp.float32),
            pltpu.VMEM((_CB, _D), jnp.float32),
            pltpu.VMEM((_CB * _K, _D), jnp.float32),
            pltpu.VMEM((16,), jnp.float32),
            pltpu.SemaphoreType.DMA,
        ],
    )(c3, p3, n4, W_in, W_out)
    return jnp.sum(partials) / jnp.float32(_B)
